# Initial kernel scaffold; baseline (speedup 1.0000x reference)
import jax
import jax.numpy as jnp
from jax.experimental import pallas as pl


def kernel(data, centers):
    raise NotImplementedError("write your pallas kernel here")


# speedup vs baseline: 8.6918x; 8.6918x over previous
import functools

import jax
import jax.numpy as jnp
from jax.experimental import pallas as pl

SIGMA = 1.0
C_NUM = 512
Z_CHANNELS = 64


def _vq_kernel(x_ref, c_ref, zbar_ref, soft_ref, hard_ref, sym_ref, phi_ref):
    x = x_ref[0]
    c = c_ref[...]

    g = jax.lax.dot_general(
        c, x, (((1,), (0,)), ((), ())),
        preferred_element_type=jnp.float32,
        precision=jax.lax.Precision.HIGHEST)
    cn = jnp.sum(c * c, axis=1, keepdims=True)
    xn = jnp.sum(x * x, axis=0, keepdims=True)
    d = jnp.sqrt(jnp.maximum(cn - 2.0 * g + xn, 0.0))

    mind = jnp.min(d, axis=0, keepdims=True)
    e = jnp.exp(SIGMA * (mind - d))
    phis = e / jnp.sum(e, axis=0, keepdims=True)
    phi_ref[0] = phis

    kidx = jax.lax.broadcasted_iota(jnp.int32, d.shape, 0)
    sym = jnp.min(jnp.where(d == mind, kidx, C_NUM), axis=0)
    sym_ref[0, 0] = sym

    soft = jax.lax.dot_general(
        c, phis, (((0,), (0,)), ((), ())),
        preferred_element_type=jnp.float32,
        precision=jax.lax.Precision.HIGHEST)
    soft_ref[0] = soft

    onehot = (kidx == sym[None, :]).astype(jnp.float32)
    hard = jax.lax.dot_general(
        c, onehot, (((0,), (0,)), ((), ())),
        preferred_element_type=jnp.float32,
        precision=jax.lax.Precision.HIGHEST)
    hard_ref[0] = hard

    zbar_ref[0] = soft + (hard - soft)


@jax.jit
def kernel(data, centers):
    b, c, h, w = data.shape
    t = h * w
    k = centers.shape[0]
    x3 = data.reshape(b, c, t)

    grid = (b,)
    out_shapes = (
        jax.ShapeDtypeStruct((b, c, t), jnp.float32),
        jax.ShapeDtypeStruct((b, c, t), jnp.float32),
        jax.ShapeDtypeStruct((b, c, t), jnp.float32),
        jax.ShapeDtypeStruct((b, 1, t), jnp.int32),
        jax.ShapeDtypeStruct((b, k, t), jnp.float32),
    )
    bspec = lambda r, cols: pl.BlockSpec((1, r, cols), lambda i: (i, 0, 0))
    zbar, soft, hard, sym, phis = pl.pallas_call(
        _vq_kernel,
        grid=grid,
        in_specs=[
            pl.BlockSpec((1, c, t), lambda i: (i, 0, 0)),
            pl.BlockSpec((k, c), lambda i: (0, 0)),
        ],
        out_specs=(
            bspec(c, t), bspec(c, t), bspec(c, t), bspec(1, t), bspec(k, t),
        ),
        out_shape=out_shapes,
    )(x3, centers)

    shp = lambda a, ch: a.reshape(b, ch, h, w)
    return (shp(zbar, c), shp(soft, c), shp(hard, c),
            shp(sym, 1), shp(phis, k))


# speedup vs baseline: 10.2350x; 1.1776x over previous
import functools

import jax
import jax.numpy as jnp
from jax.experimental import pallas as pl

SIGMA = 1.0
C_NUM = 512
Z_CHANNELS = 64


def _vq_kernel(x_ref, c_ref, zbar_ref, soft_ref, hard_ref, sym_ref, phi_ref):
    x = x_ref[0]
    c = c_ref[...]

    g = jax.lax.dot_general(
        c, x, (((1,), (0,)), ((), ())),
        preferred_element_type=jnp.float32,
        precision=jax.lax.Precision.HIGHEST)
    cn = jnp.sum(c * c, axis=1, keepdims=True)
    xn = jnp.sum(x * x, axis=0, keepdims=True)
    d = jnp.sqrt(jnp.maximum(cn - 2.0 * g + xn, 0.0))

    mind = jnp.min(d, axis=0, keepdims=True)
    e = jnp.exp(SIGMA * (mind - d))
    phis = e * (1.0 / jnp.sum(e, axis=0, keepdims=True))
    phi_ref[0] = phis

    kidx = jax.lax.broadcasted_iota(jnp.int32, d.shape, 0)
    sym = jnp.min(jnp.where(d == mind, kidx, C_NUM), axis=0)
    sym_ref[0, 0] = sym

    soft = jax.lax.dot_general(
        c, phis, (((0,), (0,)), ((), ())),
        preferred_element_type=jnp.float32)
    soft_ref[0] = soft

    onehot = (kidx == sym[None, :]).astype(jnp.float32)
    hard = jax.lax.dot_general(
        c, onehot, (((0,), (0,)), ((), ())),
        preferred_element_type=jnp.float32)
    hard_ref[0] = hard

    zbar_ref[0] = soft + (hard - soft)


@jax.jit
def kernel(data, centers):
    b, c, h, w = data.shape
    t = h * w
    k = centers.shape[0]
    x3 = data.reshape(b, c, t)

    grid = (b,)
    out_shapes = (
        jax.ShapeDtypeStruct((b, c, t), jnp.float32),
        jax.ShapeDtypeStruct((b, c, t), jnp.float32),
        jax.ShapeDtypeStruct((b, c, t), jnp.float32),
        jax.ShapeDtypeStruct((b, 1, t), jnp.int32),
        jax.ShapeDtypeStruct((b, k, t), jnp.float32),
    )
    bspec = lambda r, cols: pl.BlockSpec((1, r, cols), lambda i: (i, 0, 0))
    zbar, soft, hard, sym, phis = pl.pallas_call(
        _vq_kernel,
        grid=grid,
        in_specs=[
            pl.BlockSpec((1, c, t), lambda i: (i, 0, 0)),
            pl.BlockSpec((k, c), lambda i: (0, 0)),
        ],
        out_specs=(
            bspec(c, t), bspec(c, t), bspec(c, t), bspec(1, t), bspec(k, t),
        ),
        out_shape=out_shapes,
    )(x3, centers)

    shp = lambda a, ch: a.reshape(b, ch, h, w)
    return (shp(zbar, c), shp(soft, c), shp(hard, c),
            shp(sym, 1), shp(phis, k))


# speedup vs baseline: 17.5288x; 1.7126x over previous
import jax
import jax.numpy as jnp
from jax.experimental import pallas as pl

SIGMA = 1.0
C_NUM = 512
Z_CHANNELS = 64
TILE = 512


def _vq_kernel(x_ref, c_ref, zbar_ref, soft_ref, hard_ref, sym_ref, phi_ref):
    x = x_ref[...]
    c = c_ref[...]

    g = jax.lax.dot_general(
        x, c, (((1,), (1,)), ((), ())),
        preferred_element_type=jnp.float32,
        precision=jax.lax.Precision.HIGHEST)
    xn = jnp.sum(x * x, axis=1, keepdims=True)
    cn = jnp.sum(c * c, axis=1, keepdims=True).T
    d = jnp.sqrt(jnp.maximum(xn - 2.0 * g + cn, 0.0))

    mind = jnp.min(d, axis=1, keepdims=True)
    e = jnp.exp(SIGMA * (mind - d))
    phis = e * (1.0 / jnp.sum(e, axis=1, keepdims=True))
    phi_ref[...] = phis

    kidx = jax.lax.broadcasted_iota(jnp.int32, d.shape, 1)
    sym = jnp.min(jnp.where(d == mind, kidx, C_NUM), axis=1)
    sym_ref[0, 0] = sym

    soft = jax.lax.dot_general(
        phis, c, (((1,), (0,)), ((), ())),
        preferred_element_type=jnp.float32)
    soft_ref[...] = soft

    onehot = (kidx == sym[:, None]).astype(jnp.float32)
    hard = jax.lax.dot_general(
        onehot, c, (((1,), (0,)), ((), ())),
        preferred_element_type=jnp.float32)
    hard_ref[...] = hard

    zbar_ref[...] = soft + (hard - soft)


@jax.jit
def kernel(data, centers):
    b, c, h, w = data.shape
    n = b * h * w
    k = centers.shape[0]
    nb = n // TILE
    x = jnp.transpose(data, (0, 2, 3, 1)).reshape(n, c)

    out_shapes = (
        jax.ShapeDtypeStruct((n, c), jnp.float32),
        jax.ShapeDtypeStruct((n, c), jnp.float32),
        jax.ShapeDtypeStruct((n, c), jnp.float32),
        jax.ShapeDtypeStruct((nb, 1, TILE), jnp.int32),
        jax.ShapeDtypeStruct((n, k), jnp.float32),
    )
    tok = lambda cols: pl.BlockSpec((TILE, cols), lambda i: (i, 0))
    zbar, soft, hard, sym, phis = pl.pallas_call(
        _vq_kernel,
        grid=(nb,),
        in_specs=[
            tok(c),
            pl.BlockSpec((k, c), lambda i: (0, 0)),
        ],
        out_specs=(
            tok(c), tok(c), tok(c),
            pl.BlockSpec((1, 1, TILE), lambda i: (i, 0, 0)),
            tok(k),
        ),
        out_shape=out_shapes,
    )(x, centers)

    def to_bchw(a, ch):
        return jnp.transpose(a.reshape(b, h, w, ch), (0, 3, 1, 2))

    return (to_bchw(zbar, c), to_bchw(soft, c), to_bchw(hard, c),
            sym.reshape(b, 1, h, w), to_bchw(phis, k))


# speedup vs baseline: 20.1098x; 1.1472x over previous
import jax
import jax.numpy as jnp
from jax.experimental import pallas as pl

SIGMA = 1.0
C_NUM = 512
Z_CHANNELS = 64
TILE = 576


def _vq_kernel(x_ref, c_ref, zbar_ref, soft_ref, hard_ref, sym_ref, phi_ref):
    x = x_ref[...]
    c = c_ref[...]

    xn = jnp.sum(x * x, axis=1, keepdims=True)
    cn = jnp.sum(c * c, axis=1, keepdims=True)
    ones_t = jnp.ones((x.shape[0], 1), jnp.float32)
    ones_k = jnp.ones((c.shape[0], 1), jnp.float32)
    x_aug = jnp.concatenate([x, ones_t, xn], axis=1)
    c_aug = jnp.concatenate([-2.0 * c, cn, ones_k], axis=1)
    d2 = jax.lax.dot_general(
        x_aug, c_aug, (((1,), (1,)), ((), ())),
        preferred_element_type=jnp.float32,
        precision=jax.lax.Precision.HIGHEST)
    d = jnp.sqrt(jnp.maximum(d2, 0.0))

    mind = jnp.min(d, axis=1, keepdims=True)
    e = jnp.exp(SIGMA * (mind - d))
    phis = e * (1.0 / jnp.sum(e, axis=1, keepdims=True))
    phi_ref[...] = phis

    kidx = jax.lax.broadcasted_iota(jnp.int32, d.shape, 1)
    sym = jnp.min(jnp.where(d == mind, kidx, C_NUM), axis=1)
    sym_ref[0, 0] = sym.reshape(24, 24)

    soft = jax.lax.dot_general(
        phis, c, (((1,), (0,)), ((), ())),
        preferred_element_type=jnp.float32)
    soft_ref[...] = soft

    onehot = (kidx == sym[:, None]).astype(jnp.float32)
    hard = jax.lax.dot_general(
        onehot, c, (((1,), (0,)), ((), ())),
        preferred_element_type=jnp.float32)
    hard_ref[...] = hard

    zbar_ref[...] = soft + (hard - soft)


@jax.jit
def kernel(data, centers):
    b, c, h, w = data.shape
    n = b * h * w
    k = centers.shape[0]
    nb = n // TILE
    x = jnp.transpose(data, (0, 2, 3, 1)).reshape(n, c)

    out_shapes = (
        jax.ShapeDtypeStruct((n, c), jnp.float32),
        jax.ShapeDtypeStruct((n, c), jnp.float32),
        jax.ShapeDtypeStruct((n, c), jnp.float32),
        jax.ShapeDtypeStruct((nb, 1, h, w), jnp.int32),
        jax.ShapeDtypeStruct((n, k), jnp.float32),
    )
    tok = lambda cols: pl.BlockSpec((TILE, cols), lambda i: (i, 0))
    zbar, soft, hard, sym, phis = pl.pallas_call(
        _vq_kernel,
        grid=(nb,),
        in_specs=[
            tok(c),
            pl.BlockSpec((k, c), lambda i: (0, 0)),
        ],
        out_specs=(
            tok(c), tok(c), tok(c),
            pl.BlockSpec((1, 1, h, w), lambda i: (i, 0, 0, 0)),
            tok(k),
        ),
        out_shape=out_shapes,
    )(x, centers)

    def to_bchw(a, ch):
        return jnp.transpose(a.reshape(b, h, w, ch), (0, 3, 1, 2))

    return (to_bchw(zbar, c), to_bchw(soft, c), to_bchw(hard, c),
            sym, to_bchw(phis, k))
